# Initial kernel scaffold; baseline (speedup 1.0000x reference)
import functools

import jax
import jax.numpy as jnp
from jax import lax
from jax.experimental import pallas as pl
from jax.experimental.pallas import tpu as pltpu
from jax.experimental.pallas import tpu_sc as plsc

FLOAT_LEN = 50
D = 128
EPS = 1e-4
INT_LEN_PAD = 152
NLANE = 16
NCHUNK = D // NLANE


def _rsqrt_scalar(v):
    i = lax.bitcast_convert_type(v, jnp.int32)
    y = lax.bitcast_convert_type(jnp.int32(0x5F3759DF) - (i >> 1), jnp.float32)
    for _ in range(3):
        y = y * (1.5 - 0.5 * v * y * y)
    return y


def _make_sc_kernel(B, seq):
    int_len = seq - FLOAT_LEN
    info = plsc.get_sparse_core_info()
    nworkers = info.num_cores * info.num_subcores
    bpw = B // nworkers
    fpw = bpw * FLOAT_LEN

    mesh = plsc.VectorSubcoreMesh(core_axis_name="c", subcore_axis_name="s")

    @functools.partial(
        pl.kernel,
        mesh=mesh,
        out_type=jax.ShapeDtypeStruct((B, seq, D), jnp.float32),
        scratch_types=[
            pltpu.VMEM((INT_LEN_PAD,), jnp.int32),
            pltpu.VMEM((INT_LEN_PAD, D), jnp.float32),
            pltpu.VMEM((FLOAT_LEN, D), jnp.float32),
            pltpu.VMEM((fpw,), jnp.int32),
            pltpu.VMEM((D,), jnp.float32),
            pltpu.VMEM((D,), jnp.float32),
            pltpu.VMEM((D,), jnp.float32),
            pltpu.SemaphoreType.DMA,
        ],
    )
    def sc_kernel(idx_hbm, fvals_hbm, fw_hbm, fb_hbm, rw_hbm, table_hbm,
                  out_hbm, idx_v, rows_v, fout_v, fvals_v, fw_v, fb_v, rw_v,
                  sem):
        wid = lax.axis_index("s") * info.num_cores + lax.axis_index("c")
        pltpu.sync_copy(fw_hbm, fw_v)
        pltpu.sync_copy(fb_hbm, fb_v)
        pltpu.sync_copy(rw_hbm, rw_v)
        pltpu.sync_copy(fvals_hbm.at[pl.ds(wid * fpw, fpw)], fvals_v)

        fw_c = [fw_v[pl.ds(k * NLANE, NLANE)] for k in range(NCHUNK)]
        fb_c = [fb_v[pl.ds(k * NLANE, NLANE)] for k in range(NCHUNK)]
        rw_c = [rw_v[pl.ds(k * NLANE, NLANE)] for k in range(NCHUNK)]

        def do_batch(j, _):
            b = wid * bpw + j
            pltpu.sync_copy(idx_hbm.at[b], idx_v)
            cp1 = pltpu.async_copy(
                table_hbm.at[idx_v.at[pl.ds(0, 128)]],
                rows_v.at[pl.ds(0, 128)], sem)
            cp2 = pltpu.async_copy(
                table_hbm.at[idx_v.at[pl.ds(128, INT_LEN_PAD - 128)]],
                rows_v.at[pl.ds(128, INT_LEN_PAD - 128)], sem)

            def do_float(t, _):
                x = fvals_v[j * FLOAT_LEN + t].astype(jnp.float32)
                vs = [x * fw_c[k] + fb_c[k] for k in range(NCHUNK)]
                acc = vs[0] * vs[0]
                for k in range(1, NCHUNK):
                    acc = acc + vs[k] * vs[k]
                y = _rsqrt_scalar(jnp.sum(acc) * (1.0 / D) + EPS)
                for k in range(NCHUNK):
                    fout_v[t, pl.ds(k * NLANE, NLANE)] = vs[k] * (y * rw_c[k])
                return 0

            lax.fori_loop(0, FLOAT_LEN, do_float, 0)
            cp1.wait()
            cp2.wait()

            def do_int(t, _):
                vs = [rows_v[t, pl.ds(k * NLANE, NLANE)] for k in range(NCHUNK)]
                acc = vs[0] * vs[0]
                for k in range(1, NCHUNK):
                    acc = acc + vs[k] * vs[k]
                y = _rsqrt_scalar(jnp.sum(acc) * (1.0 / D) + EPS)
                for k in range(NCHUNK):
                    rows_v[t, pl.ds(k * NLANE, NLANE)] = vs[k] * (y * rw_c[k])
                return 0

            lax.fori_loop(0, int_len, do_int, 0)

            pltpu.sync_copy(fout_v, out_hbm.at[b, pl.ds(0, FLOAT_LEN)])
            pltpu.sync_copy(rows_v.at[pl.ds(0, int_len)],
                            out_hbm.at[b, pl.ds(FLOAT_LEN, int_len)])
            return 0

        lax.fori_loop(0, bpw, do_batch, 0)

    return sc_kernel


def kernel(input_sequence, float_w, float_b, int_table, rms_weight):
    B, seq = input_sequence.shape
    seq_i = input_sequence.astype(jnp.int32)
    idx_p = jnp.pad(seq_i[:, FLOAT_LEN:],
                    ((0, 0), (0, INT_LEN_PAD - (seq - FLOAT_LEN))))
    fvals = seq_i[:, :FLOAT_LEN].reshape(-1)
    sc = _make_sc_kernel(B, seq)
    return sc(idx_p, fvals, float_w.reshape(-1), float_b, rms_weight,
              int_table)


# speedup vs baseline: 1.1996x; 1.1996x over previous
import functools

import jax
import jax.numpy as jnp
from jax import lax
from jax.experimental import pallas as pl
from jax.experimental.pallas import tpu as pltpu
from jax.experimental.pallas import tpu_sc as plsc

FLOAT_LEN = 50
FLOAT_PAD = 64
D = 128
EPS = 1e-4
INT_LEN_PAD = 152
NLANE = 16
NCHUNK = D // NLANE


def _rsqrt_newton(v):
    i = lax.bitcast_convert_type(v, jnp.int32)
    y = lax.bitcast_convert_type(jnp.int32(0x5F3759DF) - (i >> 1), jnp.float32)
    for _ in range(3):
        y = y * (1.5 - 0.5 * v * y * y)
    return y


def _pairwise_sumsq(vs):
    sq = [v * v for v in vs]
    while len(sq) > 1:
        sq = [sq[i] + sq[i + 1] for i in range(0, len(sq), 2)]
    return sq[0]


def _make_sc_kernel(B, seq):
    int_len = seq - FLOAT_LEN
    info = plsc.get_sparse_core_info()
    nworkers = info.num_cores * info.num_subcores
    bpw = B // nworkers
    fpw = bpw * FLOAT_PAD

    mesh = plsc.VectorSubcoreMesh(core_axis_name="c", subcore_axis_name="s")

    @functools.partial(
        pl.kernel,
        mesh=mesh,
        out_type=jax.ShapeDtypeStruct((B, seq, D), jnp.float32),
        compiler_params=pltpu.CompilerParams(use_tc_tiling_on_sc=False,
                                             needs_layout_passes=False),
        scratch_types=[
            pltpu.VMEM((INT_LEN_PAD,), jnp.int32),
            pltpu.VMEM((INT_LEN_PAD, D), jnp.float32),
            pltpu.VMEM((FLOAT_PAD, D), jnp.float32),
            pltpu.VMEM((fpw,), jnp.int32),
            pltpu.VMEM((D,), jnp.float32),
            pltpu.VMEM((D,), jnp.float32),
            pltpu.VMEM((D,), jnp.float32),
            pltpu.SemaphoreType.DMA,
        ],
    )
    def sc_kernel(idx_hbm, fvals_hbm, fw_hbm, fb_hbm, rw_hbm, table_hbm,
                  out_hbm, idx_v, rows_v, fout_v, fvals_v, fw_v, fb_v, rw_v,
                  sem):
        wid = lax.axis_index("s") * info.num_cores + lax.axis_index("c")
        pltpu.sync_copy(fw_hbm, fw_v)
        pltpu.sync_copy(fb_hbm, fb_v)
        pltpu.sync_copy(rw_hbm, rw_v)
        pltpu.sync_copy(fvals_hbm.at[pl.ds(wid * fpw, fpw)], fvals_v)

        fw_c = [fw_v[pl.ds(k * NLANE, NLANE)] for k in range(NCHUNK)]
        fb_c = [fb_v[pl.ds(k * NLANE, NLANE)] for k in range(NCHUNK)]
        rw_c = [rw_v[pl.ds(k * NLANE, NLANE)] for k in range(NCHUNK)]
        fwrw_c = [fw_c[k] * rw_c[k] for k in range(NCHUNK)]
        fbrw_c = [fb_c[k] * rw_c[k] for k in range(NCHUNK)]

        sww = jnp.sum(_pairwise_sumsq(fw_c))
        sbb = jnp.sum(_pairwise_sumsq(fb_c))
        swb_acc = fw_c[0] * fb_c[0]
        for k in range(1, NCHUNK):
            swb_acc = swb_acc + fw_c[k] * fb_c[k]
        swb2 = 2.0 * jnp.sum(swb_acc)

        def do_batch(j, _):
            b = wid * bpw + j
            pltpu.sync_copy(idx_hbm.at[b], idx_v)
            cp1 = pltpu.async_copy(
                table_hbm.at[idx_v.at[pl.ds(0, 128)]],
                rows_v.at[pl.ds(0, 128)], sem)
            cp2 = pltpu.async_copy(
                table_hbm.at[idx_v.at[pl.ds(128, INT_LEN_PAD - 128)]],
                rows_v.at[pl.ds(128, INT_LEN_PAD - 128)], sem)

            for g in range(FLOAT_PAD // NLANE):
                xv = fvals_v[pl.ds(j * FLOAT_PAD + g * NLANE,
                                   NLANE)].astype(jnp.float32)
                var = (sww * xv * xv + swb2 * xv + sbb) * (1.0 / D) + EPS
                y16 = _rsqrt_newton(var)
                xy = xv * y16
                for i in range(NLANE):
                    a = xy[i]
                    c = y16[i]
                    for k in range(NCHUNK):
                        fout_v[g * NLANE + i, pl.ds(k * NLANE, NLANE)] = (
                            a * fwrw_c[k] + c * fbrw_c[k])

            cp1.wait()
            cp2.wait()

            def do_int(t, _):
                vs = [rows_v[t, pl.ds(k * NLANE, NLANE)]
                      for k in range(NCHUNK)]
                s = jnp.sum(_pairwise_sumsq(vs)) * (1.0 / D) + EPS
                y = _rsqrt_newton(s)
                for k in range(NCHUNK):
                    rows_v[t, pl.ds(k * NLANE, NLANE)] = vs[k] * (y * rw_c[k])
                return 0

            lax.fori_loop(0, int_len, do_int, 0, unroll=2)

            pltpu.sync_copy(fout_v.at[pl.ds(0, FLOAT_LEN)],
                            out_hbm.at[b, pl.ds(0, FLOAT_LEN)])
            pltpu.sync_copy(rows_v.at[pl.ds(0, int_len)],
                            out_hbm.at[b, pl.ds(FLOAT_LEN, int_len)])
            return 0

        lax.fori_loop(0, bpw, do_batch, 0)

    return sc_kernel


def kernel(input_sequence, float_w, float_b, int_table, rms_weight):
    B, seq = input_sequence.shape
    seq_i = input_sequence.astype(jnp.int32)
    idx_p = jnp.pad(seq_i[:, FLOAT_LEN:],
                    ((0, 0), (0, INT_LEN_PAD - (seq - FLOAT_LEN))))
    fvals = jnp.pad(seq_i[:, :FLOAT_LEN],
                    ((0, 0), (0, FLOAT_PAD - FLOAT_LEN))).reshape(-1)
    sc = _make_sc_kernel(B, seq)
    return sc(idx_p, fvals, float_w.reshape(-1), float_b, rms_weight,
              int_table)
